# baseline (speedup 1.0000x reference)
import functools

import jax
import jax.numpy as jnp
from jax import lax
from jax.experimental import pallas as pl
from jax.experimental.pallas import tpu as pltpu
from jax.experimental.pallas import tpu_sc as plsc

N = 10000
E = 320000
D_IN = 128
HID = 64
NUM_LAYERS = 5

NC = 2
NS = 16
B = 128
K = 80
EPT = K * B
EPAD = NC * NS * EPT - E
NROWS = N + 8
NBUF = 6

ROWS_A = 624
TAIL0 = ROWS_A * NS
TAIL = N - TAIL0


@functools.lru_cache(maxsize=None)
def _make_sc_agg(D):
  LEAD = NBUF - 2
  mesh = plsc.VectorSubcoreMesh(
      core_axis_name="c", subcore_axis_name="s", num_cores=NC)

  @functools.partial(
      pl.kernel,
      mesh=mesh,
      compiler_params=pltpu.CompilerParams(use_tc_tiling_on_sc=False),
      out_type=jax.ShapeDtypeStruct((NC, N, D), jnp.float32),
      scratch_types=[
          pltpu.VMEM((K, B), jnp.int32),
          pltpu.VMEM((K, B), jnp.int32),
          [pltpu.VMEM((B, D), jnp.float32) for _ in range(NBUF)],
          pltpu.VMEM_SHARED((NROWS, D), jnp.float32),
          [pltpu.SemaphoreType.DMA for _ in range(NBUF)],
          [pltpu.SemaphoreType.DMA for _ in range(NBUF)],
      ],
  )
  def sc_agg(ei, x_hbm, zeros_hbm, out,
             src_idx, dst_idx, rows, acc, gsem, ssem):
    c = lax.axis_index("c")
    s = lax.axis_index("s")
    r0 = s * ROWS_A

    pltpu.sync_copy(ei.at[0, c, s], src_idx)
    pltpu.sync_copy(ei.at[1, c, s], dst_idx)
    pltpu.sync_copy(zeros_hbm.at[pl.ds(r0, ROWS_A)],
                    acc.at[pl.ds(r0, ROWS_A)])

    @pl.when(s == NS - 1)
    def _():
      pltpu.sync_copy(zeros_hbm.at[pl.ds(TAIL0, TAIL)],
                      acc.at[pl.ds(TAIL0, TAIL)])

    plsc.subcore_barrier()

    def gather(j, b):
      pltpu.async_copy(x_hbm.at[src_idx.at[j]], rows[b], gsem[b])

    def gather_wait(j, b):
      pltpu.make_async_copy(x_hbm.at[src_idx.at[j]], rows[b], gsem[b]).wait()

    def scatter(j, b):
      pltpu.async_copy(rows[b], acc.at[dst_idx.at[j]], ssem[b], add=True)

    def scatter_wait(j, b):
      pltpu.make_async_copy(rows[b], acc.at[dst_idx.at[j]], ssem[b]).wait()

    for j in range(LEAD):
      gather(j, j % NBUF)

    G = K // NBUF

    def body(i, carry):
      j0 = NBUF * i
      for b in range(NBUF):
        j = j0 + b
        gather_wait(j, b)
        scatter(j, b)

        @pl.when(j >= 2)
        def _():
          scatter_wait(j - 2, (b - 2) % NBUF)

        @pl.when(j + LEAD < K)
        def _():
          gather(j + LEAD, (b + LEAD) % NBUF)

      return carry

    lax.fori_loop(0, G, body, 0)

    g_issued = min(NBUF * G - 1 + LEAD, K - 1)
    s_waited = NBUF * G - 3
    for j in range(NBUF * G, K):
      while g_issued < j:
        nxt = g_issued + 1
        if nxt - NBUF > s_waited:
          scatter_wait(nxt - NBUF, (nxt - NBUF) % NBUF)
          s_waited = nxt - NBUF
        gather(nxt, nxt % NBUF)
        g_issued = nxt
      gather_wait(j, j % NBUF)
      scatter(j, j % NBUF)
    for j in range(max(0, s_waited + 1), K):
      scatter_wait(j, j % NBUF)

    plsc.subcore_barrier()
    pltpu.sync_copy(acc.at[pl.ds(r0, ROWS_A)],
                    out.at[c, pl.ds(r0, ROWS_A)])

    @pl.when(s == NS - 1)
    def _():
      pltpu.sync_copy(acc.at[pl.ds(TAIL0, TAIL)],
                      out.at[c, pl.ds(TAIL0, TAIL)])

  return sc_agg


@functools.lru_cache(maxsize=None)
def _make_mm0():
  BN = 2000

  def body(x, w1, o):
    o[...] = jnp.dot(x[...], w1[...], preferred_element_type=jnp.float32)

  return pl.pallas_call(
      body,
      grid=(N // BN,),
      in_specs=[
          pl.BlockSpec((BN, D_IN), lambda i: (i, 0)),
          pl.BlockSpec((D_IN, HID), lambda i: (0, 0)),
      ],
      out_specs=pl.BlockSpec((BN, HID), lambda i: (i, 0)),
      out_shape=jax.ShapeDtypeStruct((N, HID), jnp.float32),
  )


@functools.lru_cache(maxsize=None)
def _make_mlp0():
  BN = 2000

  def body(p0, p1, y, b1, w2, b2, o):
    a = jnp.maximum(p0[...] + p1[...] + y[...] + b1[...], 0.0)
    z = jnp.dot(a, w2[...], preferred_element_type=jnp.float32) + b2[...]
    o[...] = jnp.maximum(z, 0.0)

  return pl.pallas_call(
      body,
      grid=(N // BN,),
      in_specs=[
          pl.BlockSpec((BN, HID), lambda i: (i, 0)),
          pl.BlockSpec((BN, HID), lambda i: (i, 0)),
          pl.BlockSpec((BN, HID), lambda i: (i, 0)),
          pl.BlockSpec((1, HID), lambda i: (0, 0)),
          pl.BlockSpec((HID, HID), lambda i: (0, 0)),
          pl.BlockSpec((1, HID), lambda i: (0, 0)),
      ],
      out_specs=pl.BlockSpec((BN, HID), lambda i: (i, 0)),
      out_shape=jax.ShapeDtypeStruct((N, HID), jnp.float32),
  )


@functools.lru_cache(maxsize=None)
def _make_mlp(D):
  BN = 2000

  def body(p0, p1, x, w1, b1, w2, b2, o):
    h = p0[...] + p1[...] + x[...]
    a = jnp.dot(h, w1[...], preferred_element_type=jnp.float32) + b1[...]
    a = jnp.maximum(a, 0.0)
    z = jnp.dot(a, w2[...], preferred_element_type=jnp.float32) + b2[...]
    o[...] = jnp.maximum(z, 0.0)

  return pl.pallas_call(
      body,
      grid=(N // BN,),
      in_specs=[
          pl.BlockSpec((BN, D), lambda i: (i, 0)),
          pl.BlockSpec((BN, D), lambda i: (i, 0)),
          pl.BlockSpec((BN, D), lambda i: (i, 0)),
          pl.BlockSpec((D, HID), lambda i: (0, 0)),
          pl.BlockSpec((1, HID), lambda i: (0, 0)),
          pl.BlockSpec((HID, HID), lambda i: (0, 0)),
          pl.BlockSpec((1, HID), lambda i: (0, 0)),
      ],
      out_specs=pl.BlockSpec((BN, HID), lambda i: (i, 0)),
      out_shape=jax.ShapeDtypeStruct((N, HID), jnp.float32),
  )


def kernel(x, edge_index,
           W1_0, b1_0, W2_0, b2_0,
           W1_1, b1_1, W2_1, b2_1,
           W1_2, b1_2, W2_2, b2_2,
           W1_3, b1_3, W2_3, b2_3,
           W1_4, b1_4, W2_4, b2_4):
  params = [
      (W1_0, b1_0, W2_0, b2_0),
      (W1_1, b1_1, W2_1, b2_1),
      (W1_2, b1_2, W2_2, b2_2),
      (W1_3, b1_3, W2_3, b2_3),
      (W1_4, b1_4, W2_4, b2_4),
  ]
  pad = jnp.stack([jnp.zeros((EPAD,), jnp.int32),
                   jnp.full((EPAD,), N, jnp.int32)])
  ei64 = jnp.concatenate([edge_index, pad], axis=1).reshape(
      2, NC, NS, K, B)
  zeros64 = jnp.zeros((N, HID), dtype=jnp.float32)
  sc64 = _make_sc_agg(HID)

  for i, (W1, b1, W2, b2) in enumerate(params):
    if i == 0:
      y = _make_mm0()(x, W1)
      p = sc64(ei64, y, zeros64)
      x = _make_mlp0()(p[0], p[1], y, b1.reshape(1, HID), W2,
                       b2.reshape(1, HID))
    else:
      p = sc64(ei64, x, zeros64)
      x = _make_mlp(HID)(p[0], p[1], x, W1, b1.reshape(1, HID), W2,
                         b2.reshape(1, HID))
  return x


# speedup vs baseline: 3.0655x; 3.0655x over previous
import functools

import jax
import jax.numpy as jnp
from jax import lax
from jax.experimental import pallas as pl
from jax.experimental.pallas import tpu as pltpu
from jax.experimental.pallas import tpu_sc as plsc

N = 10000
E = 320000
D_IN = 128
HID = 64
NUM_LAYERS = 5

NC = 2
NS = 16
B = 80
K = 125
NROWS = N
NBUF = 8

ROWS_A = 624
TAIL0 = ROWS_A * NS
TAIL = N - TAIL0


@functools.lru_cache(maxsize=None)
def _make_sc_agg(D):
  LEAD = NBUF - 2
  mesh = plsc.VectorSubcoreMesh(
      core_axis_name="c", subcore_axis_name="s", num_cores=NC)

  @functools.partial(
      pl.kernel,
      mesh=mesh,
      compiler_params=pltpu.CompilerParams(use_tc_tiling_on_sc=False),
      out_type=jax.ShapeDtypeStruct((NC, N, D), jnp.float32),
      scratch_types=[
          pltpu.VMEM((K, B), jnp.int32),
          pltpu.VMEM((K, B), jnp.int32),
          [pltpu.VMEM((B, D), jnp.float32) for _ in range(NBUF)],
          pltpu.VMEM_SHARED((NROWS, D), jnp.float32),
          [pltpu.SemaphoreType.DMA for _ in range(NBUF)],
          [pltpu.SemaphoreType.DMA for _ in range(NBUF)],
      ],
  )
  def sc_agg(ei, x_hbm, zeros_hbm, out,
             src_idx, dst_idx, rows, acc, gsem, ssem):
    c = lax.axis_index("c")
    s = lax.axis_index("s")
    r0 = s * ROWS_A

    pltpu.sync_copy(ei.at[0, c, s], src_idx)
    pltpu.sync_copy(ei.at[1, c, s], dst_idx)
    pltpu.sync_copy(zeros_hbm.at[pl.ds(r0, ROWS_A)],
                    acc.at[pl.ds(r0, ROWS_A)])

    @pl.when(s == NS - 1)
    def _():
      pltpu.sync_copy(zeros_hbm.at[pl.ds(TAIL0, TAIL)],
                      acc.at[pl.ds(TAIL0, TAIL)])

    plsc.subcore_barrier()

    def gather(j, b):
      pltpu.async_copy(x_hbm.at[src_idx.at[j]], rows[b], gsem[b])

    def gather_wait(j, b):
      pltpu.make_async_copy(x_hbm.at[src_idx.at[j]], rows[b], gsem[b]).wait()

    def scatter(j, b):
      pltpu.async_copy(rows[b], acc.at[dst_idx.at[j]], ssem[b], add=True)

    def scatter_wait(j, b):
      pltpu.make_async_copy(rows[b], acc.at[dst_idx.at[j]], ssem[b]).wait()

    for j in range(LEAD):
      gather(j, j % NBUF)

    G = K // NBUF

    def body(i, carry):
      j0 = NBUF * i
      for b in range(NBUF):
        j = j0 + b
        gather_wait(j, b)
        scatter(j, b)

        @pl.when(j >= 2)
        def _():
          scatter_wait(j - 2, (b - 2) % NBUF)

        @pl.when(j + LEAD < K)
        def _():
          gather(j + LEAD, (b + LEAD) % NBUF)

      return carry

    lax.fori_loop(0, G, body, 0)

    g_issued = min(NBUF * G - 1 + LEAD, K - 1)
    s_waited = NBUF * G - 3
    for j in range(NBUF * G, K):
      while g_issued < j:
        nxt = g_issued + 1
        if nxt - NBUF > s_waited:
          scatter_wait(nxt - NBUF, (nxt - NBUF) % NBUF)
          s_waited = nxt - NBUF
        gather(nxt, nxt % NBUF)
        g_issued = nxt
      gather_wait(j, j % NBUF)
      scatter(j, j % NBUF)
    for j in range(max(0, s_waited + 1), K):
      scatter_wait(j, j % NBUF)

    plsc.subcore_barrier()
    pltpu.sync_copy(acc.at[pl.ds(r0, ROWS_A)],
                    out.at[c, pl.ds(r0, ROWS_A)])

    @pl.when(s == NS - 1)
    def _():
      pltpu.sync_copy(acc.at[pl.ds(TAIL0, TAIL)],
                      out.at[c, pl.ds(TAIL0, TAIL)])

  return sc_agg


@functools.lru_cache(maxsize=None)
def _make_mm0():
  BN = 2000

  def body(x, w1, o):
    o[...] = jnp.dot(x[...], w1[...], preferred_element_type=jnp.float32)

  return pl.pallas_call(
      body,
      grid=(N // BN,),
      in_specs=[
          pl.BlockSpec((BN, D_IN), lambda i: (i, 0)),
          pl.BlockSpec((D_IN, HID), lambda i: (0, 0)),
      ],
      out_specs=pl.BlockSpec((BN, HID), lambda i: (i, 0)),
      out_shape=jax.ShapeDtypeStruct((N, HID), jnp.float32),
  )


@functools.lru_cache(maxsize=None)
def _make_mlp0():
  BN = 2000

  def body(p0, p1, y, b1, w2, b2, o):
    a = jnp.maximum(p0[...] + p1[...] + y[...] + b1[...], 0.0)
    z = jnp.dot(a, w2[...], preferred_element_type=jnp.float32) + b2[...]
    o[...] = jnp.maximum(z, 0.0)

  return pl.pallas_call(
      body,
      grid=(N // BN,),
      in_specs=[
          pl.BlockSpec((BN, HID), lambda i: (i, 0)),
          pl.BlockSpec((BN, HID), lambda i: (i, 0)),
          pl.BlockSpec((BN, HID), lambda i: (i, 0)),
          pl.BlockSpec((1, HID), lambda i: (0, 0)),
          pl.BlockSpec((HID, HID), lambda i: (0, 0)),
          pl.BlockSpec((1, HID), lambda i: (0, 0)),
      ],
      out_specs=pl.BlockSpec((BN, HID), lambda i: (i, 0)),
      out_shape=jax.ShapeDtypeStruct((N, HID), jnp.float32),
  )


@functools.lru_cache(maxsize=None)
def _make_mlp(D):
  BN = 2000

  def body(p0, p1, x, w1, b1, w2, b2, o):
    h = p0[...] + p1[...] + x[...]
    a = jnp.dot(h, w1[...], preferred_element_type=jnp.float32) + b1[...]
    a = jnp.maximum(a, 0.0)
    z = jnp.dot(a, w2[...], preferred_element_type=jnp.float32) + b2[...]
    o[...] = jnp.maximum(z, 0.0)

  return pl.pallas_call(
      body,
      grid=(N // BN,),
      in_specs=[
          pl.BlockSpec((BN, D), lambda i: (i, 0)),
          pl.BlockSpec((BN, D), lambda i: (i, 0)),
          pl.BlockSpec((BN, D), lambda i: (i, 0)),
          pl.BlockSpec((D, HID), lambda i: (0, 0)),
          pl.BlockSpec((1, HID), lambda i: (0, 0)),
          pl.BlockSpec((HID, HID), lambda i: (0, 0)),
          pl.BlockSpec((1, HID), lambda i: (0, 0)),
      ],
      out_specs=pl.BlockSpec((BN, HID), lambda i: (i, 0)),
      out_shape=jax.ShapeDtypeStruct((N, HID), jnp.float32),
  )


def kernel(x, edge_index,
           W1_0, b1_0, W2_0, b2_0,
           W1_1, b1_1, W2_1, b2_1,
           W1_2, b1_2, W2_2, b2_2,
           W1_3, b1_3, W2_3, b2_3,
           W1_4, b1_4, W2_4, b2_4):
  params = [
      (W1_0, b1_0, W2_0, b2_0),
      (W1_1, b1_1, W2_1, b2_1),
      (W1_2, b1_2, W2_2, b2_2),
      (W1_3, b1_3, W2_3, b2_3),
      (W1_4, b1_4, W2_4, b2_4),
  ]
  ei64 = edge_index.reshape(2, NC, NS, K, B)
  zeros64 = jnp.zeros((N, HID), dtype=jnp.float32)
  sc64 = _make_sc_agg(HID)

  for i, (W1, b1, W2, b2) in enumerate(params):
    if i == 0:
      y = _make_mm0()(x, W1)
      p = sc64(ei64, y, zeros64)
      x = _make_mlp0()(p[0], p[1], y, b1.reshape(1, HID), W2,
                       b2.reshape(1, HID))
    else:
      p = sc64(ei64, x, zeros64)
      x = _make_mlp(HID)(p[0], p[1], x, W1, b1.reshape(1, HID), W2,
                         b2.reshape(1, HID))
  return x


# speedup vs baseline: 3.0827x; 1.0056x over previous
import functools

import jax
import jax.numpy as jnp
from jax import lax
from jax.experimental import pallas as pl
from jax.experimental.pallas import tpu as pltpu
from jax.experimental.pallas import tpu_sc as plsc

N = 10000
E = 320000
D_IN = 128
HID = 64
NUM_LAYERS = 5

NC = 2
NS = 16
B = 80
K = 125
NROWS = N
NBUF = 8

ROWS_A = 624
TAIL0 = ROWS_A * NS
TAIL = N - TAIL0


@functools.lru_cache(maxsize=None)
def _make_sc_agg(D):
  LEAD = NBUF - 2
  mesh = plsc.VectorSubcoreMesh(
      core_axis_name="c", subcore_axis_name="s", num_cores=NC)

  @functools.partial(
      pl.kernel,
      mesh=mesh,
      compiler_params=pltpu.CompilerParams(use_tc_tiling_on_sc=False),
      out_type=jax.ShapeDtypeStruct((NC, N, D), jnp.float32),
      scratch_types=[
          pltpu.VMEM((K, B), jnp.int32),
          pltpu.VMEM((K, B), jnp.int32),
          [pltpu.VMEM((B, D), jnp.float32) for _ in range(NBUF)],
          pltpu.VMEM_SHARED((NROWS, D), jnp.float32),
          [pltpu.SemaphoreType.DMA for _ in range(NBUF)],
          [pltpu.SemaphoreType.DMA for _ in range(NBUF)],
      ],
  )
  def sc_agg(ei, x_hbm, zeros_hbm, out,
             src_idx, dst_idx, rows, acc, gsem, ssem):
    c = lax.axis_index("c")
    s = lax.axis_index("s")
    r0 = s * ROWS_A

    def gather(j, b):
      pltpu.async_copy(x_hbm.at[src_idx.at[j]], rows[b], gsem[b])

    def gather_wait(j, b):
      pltpu.make_async_copy(x_hbm.at[src_idx.at[j]], rows[b], gsem[b]).wait()

    def scatter(j, b):
      pltpu.async_copy(rows[b], acc.at[dst_idx.at[j]], ssem[b], add=True)

    def scatter_wait(j, b):
      pltpu.make_async_copy(rows[b], acc.at[dst_idx.at[j]], ssem[b]).wait()

    pltpu.sync_copy(ei.at[0, c, s], src_idx)
    for j in range(LEAD):
      gather(j, j % NBUF)
    pltpu.sync_copy(ei.at[1, c, s], dst_idx)
    pltpu.sync_copy(zeros_hbm.at[pl.ds(r0, ROWS_A)],
                    acc.at[pl.ds(r0, ROWS_A)])

    @pl.when(s == NS - 1)
    def _():
      pltpu.sync_copy(zeros_hbm.at[pl.ds(TAIL0, TAIL)],
                      acc.at[pl.ds(TAIL0, TAIL)])

    plsc.subcore_barrier()

    G = K // NBUF

    def body(i, carry):
      j0 = NBUF * i
      for b in range(NBUF):
        j = j0 + b
        gather_wait(j, b)
        scatter(j, b)

        @pl.when(j >= 2)
        def _():
          scatter_wait(j - 2, (b - 2) % NBUF)

        @pl.when(j + LEAD < K)
        def _():
          gather(j + LEAD, (b + LEAD) % NBUF)

      return carry

    lax.fori_loop(0, G, body, 0)

    g_issued = min(NBUF * G - 1 + LEAD, K - 1)
    s_waited = NBUF * G - 3
    for j in range(NBUF * G, K):
      while g_issued < j:
        nxt = g_issued + 1
        if nxt - NBUF > s_waited:
          scatter_wait(nxt - NBUF, (nxt - NBUF) % NBUF)
          s_waited = nxt - NBUF
        gather(nxt, nxt % NBUF)
        g_issued = nxt
      gather_wait(j, j % NBUF)
      scatter(j, j % NBUF)
    for j in range(max(0, s_waited + 1), K):
      scatter_wait(j, j % NBUF)

    plsc.subcore_barrier()
    pltpu.sync_copy(acc.at[pl.ds(r0, ROWS_A)],
                    out.at[c, pl.ds(r0, ROWS_A)])

    @pl.when(s == NS - 1)
    def _():
      pltpu.sync_copy(acc.at[pl.ds(TAIL0, TAIL)],
                      out.at[c, pl.ds(TAIL0, TAIL)])

  return sc_agg


@functools.lru_cache(maxsize=None)
def _make_mm0():
  BN = 2000

  def body(x, w1, o):
    o[...] = jnp.dot(x[...], w1[...], preferred_element_type=jnp.float32)

  return pl.pallas_call(
      body,
      grid=(N // BN,),
      in_specs=[
          pl.BlockSpec((BN, D_IN), lambda i: (i, 0)),
          pl.BlockSpec((D_IN, HID), lambda i: (0, 0)),
      ],
      out_specs=pl.BlockSpec((BN, HID), lambda i: (i, 0)),
      out_shape=jax.ShapeDtypeStruct((N, HID), jnp.float32),
  )


@functools.lru_cache(maxsize=None)
def _make_mlp0():
  BN = 2000

  def body(p0, p1, y, b1, w2, b2, o):
    a = jnp.maximum(p0[...] + p1[...] + y[...] + b1[...], 0.0)
    z = jnp.dot(a, w2[...], preferred_element_type=jnp.float32) + b2[...]
    o[...] = jnp.maximum(z, 0.0)

  return pl.pallas_call(
      body,
      grid=(N // BN,),
      in_specs=[
          pl.BlockSpec((BN, HID), lambda i: (i, 0)),
          pl.BlockSpec((BN, HID), lambda i: (i, 0)),
          pl.BlockSpec((BN, HID), lambda i: (i, 0)),
          pl.BlockSpec((1, HID), lambda i: (0, 0)),
          pl.BlockSpec((HID, HID), lambda i: (0, 0)),
          pl.BlockSpec((1, HID), lambda i: (0, 0)),
      ],
      out_specs=pl.BlockSpec((BN, HID), lambda i: (i, 0)),
      out_shape=jax.ShapeDtypeStruct((N, HID), jnp.float32),
  )


@functools.lru_cache(maxsize=None)
def _make_mlp(D):
  BN = 2000

  def body(p0, p1, x, w1, b1, w2, b2, o):
    h = p0[...] + p1[...] + x[...]
    a = jnp.dot(h, w1[...], preferred_element_type=jnp.float32) + b1[...]
    a = jnp.maximum(a, 0.0)
    z = jnp.dot(a, w2[...], preferred_element_type=jnp.float32) + b2[...]
    o[...] = jnp.maximum(z, 0.0)

  return pl.pallas_call(
      body,
      grid=(N // BN,),
      in_specs=[
          pl.BlockSpec((BN, D), lambda i: (i, 0)),
          pl.BlockSpec((BN, D), lambda i: (i, 0)),
          pl.BlockSpec((BN, D), lambda i: (i, 0)),
          pl.BlockSpec((D, HID), lambda i: (0, 0)),
          pl.BlockSpec((1, HID), lambda i: (0, 0)),
          pl.BlockSpec((HID, HID), lambda i: (0, 0)),
          pl.BlockSpec((1, HID), lambda i: (0, 0)),
      ],
      out_specs=pl.BlockSpec((BN, HID), lambda i: (i, 0)),
      out_shape=jax.ShapeDtypeStruct((N, HID), jnp.float32),
  )


def kernel(x, edge_index,
           W1_0, b1_0, W2_0, b2_0,
           W1_1, b1_1, W2_1, b2_1,
           W1_2, b1_2, W2_2, b2_2,
           W1_3, b1_3, W2_3, b2_3,
           W1_4, b1_4, W2_4, b2_4):
  params = [
      (W1_0, b1_0, W2_0, b2_0),
      (W1_1, b1_1, W2_1, b2_1),
      (W1_2, b1_2, W2_2, b2_2),
      (W1_3, b1_3, W2_3, b2_3),
      (W1_4, b1_4, W2_4, b2_4),
  ]
  ei64 = edge_index.reshape(2, NC, NS, K, B)
  zeros64 = jnp.zeros((N, HID), dtype=jnp.float32)
  sc64 = _make_sc_agg(HID)

  for i, (W1, b1, W2, b2) in enumerate(params):
    if i == 0:
      y = _make_mm0()(x, W1)
      p = sc64(ei64, y, zeros64)
      x = _make_mlp0()(p[0], p[1], y, b1.reshape(1, HID), W2,
                       b2.reshape(1, HID))
    else:
      p = sc64(ei64, x, zeros64)
      x = _make_mlp(HID)(p[0], p[1], x, W1, b1.reshape(1, HID), W2,
                         b2.reshape(1, HID))
  return x
